# baseline (speedup 1.0000x reference)
import jax
import jax.numpy as jnp
from jax import lax
from jax.experimental import pallas as pl
from jax.experimental.pallas import tpu as pltpu
from jax.experimental.pallas import tpu_sc as plsc

B, L = 4096, 50
N = B * L
D_OUT = 256
NC, NS = 2, 16
NW = NC * NS
PER_W = N // NW
C = 80
NCHUNK = PER_W // C
NPAIR = NCHUNK // 2
G16 = C // 16


def _body(i0, i1, i2, i3, w_id, w_cat_in, w_br_in, w_sh_in, out_hbm,
          jid, jcat, jbr, jsh,
          bid0, bR0, tmpB0, tmpS0, jbp0, jsp0, jcp0,
          bid1, bR1, tmpB1, tmpS1, jbp1, jsp1, jcp1,
          nbB0, nbB1, nbS0, nbS1,
          cpad, shpad, bpad,
          gs0, gs1, ws0, ws1):
    cid = lax.axis_index("c")
    sid = lax.axis_index("s")
    wid = sid * NC + cid
    w_base = wid * PER_W

    zv = jnp.zeros((16,), jnp.float32)

    def build(src, dst, V, D, K, nbs, pbs, rsem, wsem):
        nblk = V // K
        nvec = D // 16
        nmine = (nblk - sid + NS - 1) // NS

        def blk(t):
            return sid + t * NS

        def fire_read(t, s):
            pltpu.async_copy(src.at[pl.ds(blk(t) * K, K)], nbs[s], rsem)

        def drain_read(t, s):
            pltpu.make_async_copy(
                src.at[pl.ds(blk(t) * K, K)], nbs[s], rsem).wait()

        def dst_view(t, s):
            return dst.at[pl.ds(cid * V + blk(t) * K, K)]

        def fire_write(t, s):
            pltpu.async_copy(pbs[s], dst_view(t, s), wsem)

        def drain_write(t, s):
            pltpu.make_async_copy(pbs[s], dst_view(t, s), wsem).wait()

        def assemble(s):
            nb, pb = nbs[s], pbs[s]

            def row(r, c2):
                for j in range(nvec):
                    pb[r, pl.ds(j * 16, 16)] = nb[r, pl.ds(j * 16, 16)]
                for j in range(nvec, 8):
                    pb[r, pl.ds(j * 16, 16)] = zv
                return c2
            lax.fori_loop(0, K, row, 0)

        @pl.when(nmine > 0)
        def _p0():
            fire_read(0, 0)

        @pl.when(nmine > 1)
        def _p1():
            fire_read(1, 1)

        def half(t, s):
            @pl.when(t < nmine)
            def _do():
                drain_read(t, s)
                assemble(s)

                @pl.when(t >= 2)
                def _dw():
                    drain_write(t - 2, s)
                fire_write(t, s)

                @pl.when(t + 2 < nmine)
                def _nx():
                    fire_read(t + 2, s)

        def step(p, c2):
            half(2 * p, 0)
            half(2 * p + 1, 1)
            return c2

        nloop = (nblk + NS - 1) // NS
        lax.fori_loop(0, (nloop + 1) // 2, step, 0)
        for s_ in (0, 1):
            t_s = 2 * ((nmine - 1 - s_) // 2) + s_

            @pl.when(nmine > s_)
            def _ep(t_s=t_s, s_=s_):
                drain_write(t_s, s_)

    pb40 = (bid0.at[pl.ds(0, 40)], bid1.at[pl.ds(0, 40)])
    build(w_cat_in, cpad, 1000, 32, 40, (nbS0, nbS1), pb40, gs0, ws0)
    build(w_sh_in, shpad, 10000, 32, 40, (nbS0, nbS1), pb40, gs0, ws0)
    build(w_br_in, bpad, 100000, 64, 40, (nbB0, nbB1), pb40, gs1, ws1)
    plsc.subcore_barrier()

    pltpu.sync_copy(i0.at[pl.ds(w_base, PER_W)], jid)
    pltpu.sync_copy(i1.at[pl.ds(w_base, PER_W)], jcat)
    pltpu.sync_copy(i2.at[pl.ds(w_base, PER_W)], jbr)
    pltpu.sync_copy(i3.at[pl.ds(w_base, PER_W)], jsh)

    side = [(bid0, bR0, tmpB0, tmpS0, jbp0, jsp0, jcp0, gs0, ws0),
            (bid1, bR1, tmpB1, tmpS1, jbp1, jsp1, jcp1, gs1, ws1)]

    def fire_gathers(g, s):
        bid, bR, tmpB, tmpS, jbp, jsp, jcp, gs, _ = side[s]
        off = g * C

        def mkidx(gg, c2):
            jcp[pl.ds(gg * 16, 16)] = jcat[pl.ds(off + gg * 16, 16)] + cid * 1000
            jsp[pl.ds(gg * 16, 16)] = jsh[pl.ds(off + gg * 16, 16)] + cid * 10000
            jbp[pl.ds(gg * 16, 16)] = jbr[pl.ds(off + gg * 16, 16)] + cid * 100000
            return c2
        lax.fori_loop(0, G16, mkidx, 0)
        pltpu.async_copy(w_id.at[jid.at[pl.ds(off, C)]], bid, gs)
        pltpu.async_copy(cpad.at[jcp], bR, gs)
        pltpu.async_copy(bpad.at[jbp], tmpB, gs)
        pltpu.async_copy(shpad.at[jsp], tmpS, gs)

    def drain_gathers(g, s):
        bid, bR, tmpB, tmpS, jbp, jsp, jcp, gs, _ = side[s]
        off = g * C
        pltpu.make_async_copy(w_id.at[jid.at[pl.ds(off, C)]], bid, gs).wait()
        pltpu.make_async_copy(cpad.at[jcp], bR, gs).wait()
        pltpu.make_async_copy(bpad.at[jbp], tmpB, gs).wait()
        pltpu.make_async_copy(shpad.at[jsp], tmpS, gs).wait()

    def assemble_fix(g, s):
        bid, bR, tmpB, tmpS, jbp, jsp, jcp, _, _ = side[s]
        off = g * C

        def row(r, c2):
            for j in range(4):
                bR[r, pl.ds(32 + j * 16, 16)] = tmpB[r, pl.ds(j * 16, 16)]
            for j in range(2):
                bR[r, pl.ds(96 + j * 16, 16)] = tmpS[r, pl.ds(j * 16, 16)]
            return c2
        lax.fori_loop(0, C, row, 0)

        acc = jnp.zeros((16,), jnp.int32)
        for gg in range(G16):
            iv = jid[pl.ds(off + gg * 16, 16)]
            acc = acc | jnp.where(iv == 0, 1, 0)
        nz = jnp.max(acc)

        @pl.when(nz > 0)
        def _fix():
            def per_group(i, c2):
                iv = jid[pl.ds(off + i * 16, 16)]
                z = iv == 0
                rows = lax.iota(jnp.int32, 16) + i * 16
                zf = jnp.zeros((16,), jnp.float32)
                for col in range(128):
                    cols = jnp.full((16,), col, jnp.int32)
                    plsc.store_scatter(bid, [rows, cols], zf, mask=z)
                return c2
            lax.fori_loop(0, G16, per_group, 0)

    def fire_writes(g, s):
        bid, bR = side[s][0], side[s][1]
        ws = side[s][8]
        base = w_base + g * C
        pltpu.async_copy(bid, out_hbm.at[pl.ds(base, C), pl.ds(0, 128)], ws)
        pltpu.async_copy(bR, out_hbm.at[pl.ds(base, C), pl.ds(128, 128)], ws)

    def drain_writes(g, s):
        bid, bR = side[s][0], side[s][1]
        ws = side[s][8]
        base = w_base + g * C
        pltpu.make_async_copy(
            bid, out_hbm.at[pl.ds(base, C), pl.ds(0, 128)], ws).wait()
        pltpu.make_async_copy(
            bR, out_hbm.at[pl.ds(base, C), pl.ds(128, 128)], ws).wait()

    fire_gathers(0, 0)
    fire_gathers(1, 1)

    def pair(i, carry):
        a = 2 * i
        b = a + 1
        drain_gathers(a, 0)
        assemble_fix(a, 0)
        fire_writes(a, 0)
        drain_gathers(b, 1)
        assemble_fix(b, 1)
        fire_writes(b, 1)
        drain_writes(a, 0)

        @pl.when(i < NPAIR - 1)
        def _n0():
            fire_gathers(a + 2, 0)
        drain_writes(b, 1)

        @pl.when(i < NPAIR - 1)
        def _n1():
            fire_gathers(b + 2, 1)
        return carry

    lax.fori_loop(0, NPAIR, pair, 0)


def _scratch():
    return [
        pltpu.VMEM((PER_W,), jnp.int32),
        pltpu.VMEM((PER_W,), jnp.int32),
        pltpu.VMEM((PER_W,), jnp.int32),
        pltpu.VMEM((PER_W,), jnp.int32),
        pltpu.VMEM((C, 128), jnp.float32),
        pltpu.VMEM((C, 128), jnp.float32),
        pltpu.VMEM((C, 128), jnp.float32),
        pltpu.VMEM((C, 128), jnp.float32),
        pltpu.VMEM((C,), jnp.int32),
        pltpu.VMEM((C,), jnp.int32),
        pltpu.VMEM((C,), jnp.int32),
        pltpu.VMEM((C, 128), jnp.float32),
        pltpu.VMEM((C, 128), jnp.float32),
        pltpu.VMEM((C, 128), jnp.float32),
        pltpu.VMEM((C, 128), jnp.float32),
        pltpu.VMEM((C,), jnp.int32),
        pltpu.VMEM((C,), jnp.int32),
        pltpu.VMEM((C,), jnp.int32),
        pltpu.VMEM((40, 64), jnp.float32),
        pltpu.VMEM((40, 64), jnp.float32),
        pltpu.VMEM((40, 32), jnp.float32),
        pltpu.VMEM((40, 32), jnp.float32),
        pltpu.MemorySpace.HBM((2000, 128), jnp.float32),
        pltpu.MemorySpace.HBM((20000, 128), jnp.float32),
        pltpu.MemorySpace.HBM((200000, 128), jnp.float32),
        pltpu.SemaphoreType.DMA,
        pltpu.SemaphoreType.DMA,
        pltpu.SemaphoreType.DMA,
        pltpu.SemaphoreType.DMA,
    ]


def kernel(attr_id, attr_category, attr_brand, attr_shop,
           W_id, W_category, W_brand, W_shop):
    ii = attr_id.astype(jnp.int32).reshape(N)
    ic = attr_category.astype(jnp.int32).reshape(N)
    ib = attr_brand.astype(jnp.int32).reshape(N)
    ish = attr_shop.astype(jnp.int32).reshape(N)
    k = pl.kernel(
        _body,
        out_type=jax.ShapeDtypeStruct((N, D_OUT), jnp.float32),
        mesh=plsc.VectorSubcoreMesh(core_axis_name="c", subcore_axis_name="s"),
        compiler_params=pltpu.CompilerParams(needs_layout_passes=False),
        scratch_types=_scratch(),
    )
    out = k(ii, ic, ib, ish, W_id, W_category, W_brand, W_shop)
    return out.reshape(B, L, D_OUT)
